# baseline (speedup 1.0000x reference)
import functools

import jax
import jax.numpy as jnp
from jax import lax
from jax.experimental import pallas as pl
from jax.experimental.pallas import tpu as pltpu
from jax.experimental.pallas import tpu_sc as plsc

_INFO = plsc.get_sparse_core_info()
_NC = _INFO.num_cores
_NS = _INFO.num_subcores
_NW = _NC * _NS

_NBLOCKS = 32
_BLOCK = 16 * 512 * 64 * 64 // _NBLOCKS

_mesh = plsc.VectorSubcoreMesh(core_axis_name="c", subcore_axis_name="s")


@functools.partial(
    pl.kernel,
    out_type=jax.ShapeDtypeStruct((_NBLOCKS, _BLOCK), jnp.float32),
    mesh=_mesh,
    scratch_types=[pltpu.SemaphoreType.DMA],
)
def _flip_copy(x_hbm, out_hbm, sem):
    wid = lax.axis_index("s") * _NC + lax.axis_index("c")
    src = wid
    dst = jnp.bitwise_xor(wid, 1)
    pltpu.async_copy(x_hbm.at[src], out_hbm.at[dst], sem).wait()


def kernel(x):
    n, c, h, w = x.shape
    x2 = x.reshape(_NBLOCKS, _BLOCK)
    y2 = _flip_copy(x2)
    return y2.reshape(n, c, h, w)


# speedup vs baseline: 7.1455x; 7.1455x over previous
import functools

import jax
import jax.numpy as jnp
from jax import lax
from jax.experimental import pallas as pl
from jax.experimental.pallas import tpu as pltpu
from jax.experimental.pallas import tpu_sc as plsc

_INFO = plsc.get_sparse_core_info()
_NC = _INFO.num_cores
_NS = _INFO.num_subcores
_NW = _NC * _NS

_NBLOCKS = 32
_BLOCK = 16 * 512 * 64 * 64 // _NBLOCKS
_CHUNK = 32768
_NCHUNK = _BLOCK // _CHUNK

_mesh = plsc.VectorSubcoreMesh(core_axis_name="c", subcore_axis_name="s")


@functools.partial(
    pl.kernel,
    out_type=jax.ShapeDtypeStruct((_NBLOCKS, _NCHUNK, _CHUNK), jnp.float32),
    mesh=_mesh,
    scratch_types=[
        pltpu.VMEM_SHARED((_NS, 2, _CHUNK), jnp.float32),
        pltpu.SemaphoreType.DMA,
        pltpu.SemaphoreType.DMA,
        pltpu.SemaphoreType.DMA,
        pltpu.SemaphoreType.DMA,
    ],
)
def _flip_copy(x_hbm, out_hbm, spmem, in0, in1, ot0, ot1):
    sid = lax.axis_index("s")
    wid = sid * _NC + lax.axis_index("c")
    src = wid
    dst = jnp.bitwise_xor(wid, 1)

    bufs = (spmem.at[sid, 0], spmem.at[sid, 1])
    in_sems = (in0, in1)
    out_sems = (ot0, ot1)
    in_cp = [None, None]
    out_cp = [None, None]

    def start_fetch(i):
        b = i % 2
        if out_cp[b] is not None:
            out_cp[b].wait()
        in_cp[b] = pltpu.async_copy(x_hbm.at[src, i], bufs[b], in_sems[b])

    start_fetch(0)
    for i in range(_NCHUNK):
        if i + 1 < _NCHUNK:
            start_fetch(i + 1)
        b = i % 2
        in_cp[b].wait()
        out_cp[b] = pltpu.async_copy(bufs[b], out_hbm.at[dst, i], out_sems[b])

    out_cp[(_NCHUNK - 2) % 2].wait()
    out_cp[(_NCHUNK - 1) % 2].wait()


def kernel(x):
    n, c, h, w = x.shape
    x3 = x.reshape(_NBLOCKS, _NCHUNK, _CHUNK)
    y3 = _flip_copy(x3)
    return y3.reshape(n, c, h, w)
